# baseline (speedup 1.0000x reference)
import functools

import jax
import jax.numpy as jnp
from jax import lax
from jax.experimental import pallas as pl
from jax.experimental.pallas import tpu as pltpu
from jax.experimental.pallas import tpu_sc as plsc

N_CORES = 2
N_SUB = 16
LANES = 16
IDXW = 128
GIDXW = 1024
STREAMS = 8
WCHUNK = 368
N_PAD = 100096
ROW_R = 4352


def _sc_mesh():
    return plsc.VectorSubcoreMesh(
        core_axis_name="c", subcore_axis_name="s",
        num_cores=N_CORES, num_subcores=N_SUB)


@functools.lru_cache(maxsize=None)
def _make_deg(n_pad, e_pad):
    epw = e_pad // (N_CORES * N_SUB)
    assert epw % (8 * IDXW) == 0
    n_win = epw // (8 * IDXW)
    rps = n_pad // N_SUB
    assert rps % 8 == 0 and n_pad % N_SUB == 0

    def body(dst_hbm, out0_hbm, out1_hbm, ones_v, dstv, zbuf, acc, sem):
        c = lax.axis_index("c")
        s = lax.axis_index("s")
        def fill_ones(i, _):
            ones_v[pl.ds(i * LANES, LANES)] = jnp.ones((LANES,), jnp.float32)
            return 0
        lax.fori_loop(0, IDXW // LANES, fill_ones, 0)
        def fill_z(i, _):
            zbuf[pl.ds(i * LANES, LANES)] = jnp.zeros((LANES,), jnp.float32)
            return 0
        lax.fori_loop(0, rps // LANES, fill_z, 0)
        pltpu.sync_copy(zbuf, acc.at[pl.ds(s * rps, rps)])
        plsc.subcore_barrier()
        wid = c * N_SUB + s
        row0 = wid * (epw // IDXW)
        def win(w, _):
            pltpu.sync_copy(dst_hbm.at[pl.ds(row0 + w * 8, 8)], dstv)
            descs = [pltpu.async_copy(ones_v, acc.at[dstv.at[j]], sem, add=True)
                     for j in range(8)]
            for d in descs:
                d.wait()
            return 0
        lax.fori_loop(0, n_win, win, 0)
        plsc.subcore_barrier()
        pltpu.sync_copy(acc.at[pl.ds(s * rps, rps)], zbuf)

        @pl.when(c == 0)
        def _():
            pltpu.sync_copy(zbuf, out0_hbm.at[pl.ds(s * rps, rps)])

        @pl.when(c == 1)
        def _():
            pltpu.sync_copy(zbuf, out1_hbm.at[pl.ds(s * rps, rps)])

    return pl.kernel(
        body,
        out_type=(jax.ShapeDtypeStruct((n_pad,), jnp.float32),
                  jax.ShapeDtypeStruct((n_pad,), jnp.float32)),
        mesh=_sc_mesh(),
        scratch_types=[
            pltpu.VMEM((IDXW,), jnp.float32),
            pltpu.VMEM((8, IDXW), jnp.int32),
            pltpu.VMEM((rps,), jnp.float32),
            pltpu.VMEM_SHARED((n_pad,), jnp.float32),
            pltpu.SemaphoreType.DMA,
        ],
        compiler_params=pltpu.CompilerParams(use_tc_tiling_on_sc=False),
    )


@functools.lru_cache(maxsize=None)
def _make_agg(n_blocks, n_pad, e_pad):
    eps = e_pad // N_SUB
    win_e = GIDXW
    assert eps % win_e == 0
    n_win = eps // win_e
    rps = n_pad // N_SUB
    assert rps % WCHUNK == 0 and WCHUNK % 8 == 0
    packed_rows = n_pad * n_blocks * LANES // 128

    def body(z_hbm, src_hbm, dst_hbm, out_hbm,
             srcv, dstv, sidx, rows, zbuf, acc, gsem, ssem):
        zf = z_hbm
        of = out_hbm
        c = lax.axis_index("c")
        s = lax.axis_index("s")

        def fill_z(i, _):
            zbuf[i] = jnp.zeros((LANES,), jnp.float32)
            return 0
        lax.fori_loop(0, WCHUNK, fill_z, 0)

        def process(b):
            def zero(k, _):
                pltpu.sync_copy(zbuf, acc.at[pl.ds(s * rps + k * WCHUNK, WCHUNK)])
                return 0
            lax.fori_loop(0, rps // WCHUNK, zero, 0)
            plsc.subcore_barrier()
            row0 = s * (eps // GIDXW)

            def win(w, _):
                r = row0 + w
                pltpu.sync_copy(src_hbm.at[pl.ds(r, 1)], srcv)
                pltpu.sync_copy(dst_hbm.at[pl.ds(r, 1)], dstv)
                for k in range(GIDXW // LANES):
                    sl = pl.ds(k * LANES, LANES)
                    sidx[0, sl] = srcv[0, sl] * n_blocks + b
                pltpu.async_copy(zf.at[sidx.at[0]], rows, gsem).wait()
                pltpu.async_copy(rows, acc.at[dstv.at[0]], ssem,
                                 add=True).wait()
                return 0
            lax.fori_loop(0, n_win, win, 0)
            plsc.subcore_barrier()

            def wb(k, _):
                r0 = s * rps + k * WCHUNK
                pltpu.sync_copy(acc.at[pl.ds(r0, WCHUNK)], rows.at[pl.ds(0, WCHUNK)])
                pltpu.sync_copy(rows.at[pl.ds(0, WCHUNK)],
                                of.at[pl.ds(r0, WCHUNK), b])
                return 0
            lax.fori_loop(0, rps // WCHUNK, wb, 0)
            plsc.subcore_barrier()

        for c_val in range(N_CORES):
            blocks = list(range(c_val, n_blocks, N_CORES))
            if not blocks:
                continue

            @pl.when(c == c_val)
            def _(blocks=blocks):
                for b in blocks:
                    process(b)

    del packed_rows
    return pl.kernel(
        body,
        out_type=jax.ShapeDtypeStruct((n_pad, n_blocks, LANES), jnp.float32),
        mesh=_sc_mesh(),
        scratch_types=[
            pltpu.VMEM((1, GIDXW), jnp.int32),
            pltpu.VMEM((1, GIDXW), jnp.int32),
            pltpu.VMEM((1, GIDXW), jnp.int32),
            pltpu.VMEM((GIDXW, LANES), jnp.float32),
            pltpu.VMEM((WCHUNK, LANES), jnp.float32),
            pltpu.VMEM_SHARED((n_pad, LANES), jnp.float32),
            pltpu.SemaphoreType.DMA,
            pltpu.SemaphoreType.DMA,
        ],
        compiler_params=pltpu.CompilerParams(use_tc_tiling_on_sc=False),
    )


def _sc_agg(zp, src2, dst2, nb, e_pad):
    zf = zp.reshape(N_PAD * nb, LANES)
    src2g = src2.reshape(e_pad // GIDXW, GIDXW)
    dst2g = dst2.reshape(e_pad // GIDXW, GIDXW)
    y = _make_agg(nb, N_PAD, e_pad)(zf, src2g, dst2g)
    return y.reshape(N_PAD * nb * LANES // 128, 128)


GRID = N_PAD // ROW_R


def _leaky(v):
    return jnp.where(v >= 0, v, 0.01 * v)


def _pspec(d):
    rows = ROW_R * d // 128
    return pl.BlockSpec((rows, 128), lambda i: (i, 0))


def _rspec(w):
    return pl.BlockSpec((ROW_R, w), lambda i: (i, 0))


def _fspec(shape):
    return pl.BlockSpec(shape, lambda i: (0, 0))


def _packed_struct(d):
    return jax.ShapeDtypeStruct((N_PAD * d // 128, 128), jnp.float32)


def _unpack(ref, d):
    s = 128 // d
    y = ref[...]
    parts = [y[:, q * d:(q + 1) * d].reshape(ROW_R // s, 1, d) for q in range(s)]
    return jnp.concatenate(parts, axis=1).reshape(ROW_R, d)


def _pack(v, d):
    s = 128 // d
    t = v.reshape(ROW_R // s, s, d)
    return jnp.concatenate(
        [t[:, q:q + 1, :].reshape(ROW_R // s, d) for q in range(s)], axis=-1)


def _tc1(xp, W1, d0, d1):
    dW1 = W1.shape[1]

    def body(x_r, d0_r, d1_r, w_r, z_r, dinv_r):
        dinv = lax.rsqrt(d0_r[...] + d1_r[...] + 1.0)
        dinv_r[...] = dinv
        z_r[...] = _pack(dinv * jnp.dot(x_r[...], w_r[...],
                                        preferred_element_type=jnp.float32), dW1)

    return pl.pallas_call(
        body, grid=(GRID,),
        in_specs=[_rspec(xp.shape[1]), _rspec(1), _rspec(1), _fspec(W1.shape)],
        out_specs=[_pspec(dW1), _rspec(1)],
        out_shape=[_packed_struct(dW1),
                   jax.ShapeDtypeStruct((N_PAD, 1), jnp.float32)],
    )(xp, d0, d1, W1)


def _tc2(y1, z1, dinv, b1, d):
    def body(y_r, z_r, dv_r, b_r, o_r):
        h = _leaky(dv_r[...] * (_unpack(y_r, d) + _unpack(z_r, d)) + b_r[...])
        o_r[...] = _pack(dv_r[...] * h, d)

    return pl.pallas_call(
        body, grid=(GRID,),
        in_specs=[_pspec(d), _pspec(d), _rspec(1), _fspec(b1.shape)],
        out_specs=_pspec(d), out_shape=_packed_struct(d),
    )(y1, z1, dinv, b1)


def _tc3(y2, z2, dinv, W2, b2, W3, d_in):
    d_out = W3.shape[1]

    def body(y_r, z_r, dv_r, w2_r, b2_r, w3_r, o_r):
        t = dv_r[...] * (_unpack(y_r, d_in) + _unpack(z_r, d_in))
        h = _leaky(jnp.dot(t, w2_r[...], preferred_element_type=jnp.float32)
                   + b2_r[...])
        o_r[...] = _pack(dv_r[...] * jnp.dot(h, w3_r[...],
                                             preferred_element_type=jnp.float32),
                         d_out)

    return pl.pallas_call(
        body, grid=(GRID,),
        in_specs=[_pspec(d_in), _pspec(d_in), _rspec(1),
                  _fspec(W2.shape), _fspec(b2.shape), _fspec(W3.shape)],
        out_specs=_pspec(d_out), out_shape=_packed_struct(d_out),
    )(y2, z2, dinv, W2, b2, W3)


def _tc4(y3, z3, dinv, b3, W4, d_in):
    d_out = W4.shape[1]

    def body(y_r, z_r, dv_r, b3_r, w4_r, o_r):
        h = _leaky(dv_r[...] * (_unpack(y_r, d_in) + _unpack(z_r, d_in))
                   + b3_r[...])
        o_r[...] = _pack(dv_r[...] * jnp.dot(h, w4_r[...],
                                             preferred_element_type=jnp.float32),
                         d_out)

    return pl.pallas_call(
        body, grid=(GRID,),
        in_specs=[_pspec(d_in), _pspec(d_in), _rspec(1),
                  _fspec(b3.shape), _fspec(W4.shape)],
        out_specs=_pspec(d_out), out_shape=_packed_struct(d_out),
    )(y3, z3, dinv, b3, W4)


def _tc5(y4, z4, dinv, b4, Wl, bl, d_in):
    d_out = Wl.shape[1]

    def body(y_r, z_r, dv_r, b4_r, wl_r, bl_r, o_r):
        t = (dv_r[...] * (_unpack(y_r, d_in) + _unpack(z_r, d_in)))
        h = _leaky(t[:, :b4_r.shape[1]] + b4_r[...])
        o_r[...] = jnp.maximum(
            jnp.dot(h, wl_r[...], preferred_element_type=jnp.float32)
            + bl_r[...], 0.0)

    return pl.pallas_call(
        body, grid=(GRID,),
        in_specs=[_pspec(d_in), _pspec(d_in), _rspec(1),
                  _fspec(b4.shape), _fspec(Wl.shape), _fspec(bl.shape)],
        out_specs=_rspec(d_out),
        out_shape=jax.ShapeDtypeStruct((N_PAD, d_out), jnp.float32),
    )(y4, z4, dinv, b4, Wl, bl)


def kernel(x, edge_index, W1, b1, W2, b2, W3, b3, W4, b4, Wl, bl):
    n = x.shape[0]
    e = edge_index.shape[1]
    assert n <= N_PAD

    e_unit = N_CORES * N_SUB * STREAMS * IDXW
    e_pad = ((e + e_unit - 1) // e_unit) * e_unit

    src = edge_index[0].astype(jnp.int32)
    dst = edge_index[1].astype(jnp.int32)
    if e_pad != e:
        pad = e_pad - e
        pad_dst = n + jnp.arange(pad, dtype=jnp.int32) % (N_PAD - n)
        src = jnp.concatenate([src, jnp.full((pad,), n - 1, jnp.int32)])
        dst = jnp.concatenate([dst, pad_dst])
    src2 = src.reshape(e_pad // IDXW, IDXW)
    dst2 = dst.reshape(e_pad // IDXW, IDXW)

    deg0, deg1 = _make_deg(N_PAD, e_pad)(dst2)
    xp = jnp.pad(x, ((0, N_PAD - n), (0, 0)))

    b1r, b2r, b3r, b4r, blr = (v.reshape(1, -1) for v in (b1, b2, b3, b4, bl))

    W4p = jnp.pad(W4, ((0, 0), (0, 64 - W4.shape[1])))

    z1, dinv = _tc1(xp, W1, deg0[:, None], deg1[:, None])
    y1 = _sc_agg(z1, src2, dst2, 2, e_pad)
    z2 = _tc2(y1, z1, dinv, b1r, 32)
    y2 = _sc_agg(z2, src2, dst2, 2, e_pad)
    z3 = _tc3(y2, z2, dinv, W2, b2r, W3, 32)
    y3 = _sc_agg(z3, src2, dst2, 4, e_pad)
    z4 = _tc4(y3, z3, dinv, b3r, W4p, 64)
    y4 = _sc_agg(z4, src2, dst2, 4, e_pad)
    return _tc5(y4, z4, dinv, b4r, Wl, blr, 64)[:n]


# speedup vs baseline: 1.0220x; 1.0220x over previous
import functools

import jax
import jax.numpy as jnp
from jax import lax
from jax.experimental import pallas as pl
from jax.experimental.pallas import tpu as pltpu
from jax.experimental.pallas import tpu_sc as plsc

N_CORES = 2
N_SUB = 16
LANES = 16
IDXW = 128
GIDXW = 512
STREAMS = 8
WCHUNK = 368
N_PAD = 100096
ROW_R = 4352


def _sc_mesh():
    return plsc.VectorSubcoreMesh(
        core_axis_name="c", subcore_axis_name="s",
        num_cores=N_CORES, num_subcores=N_SUB)


@functools.lru_cache(maxsize=None)
def _make_deg(n_pad, e_pad):
    epw = e_pad // (N_CORES * N_SUB)
    assert epw % (8 * IDXW) == 0
    n_win = epw // (8 * IDXW)
    rps = n_pad // N_SUB
    assert rps % 8 == 0 and n_pad % N_SUB == 0

    def body(dst_hbm, out0_hbm, out1_hbm, ones_v, dstv, zbuf, acc, sem):
        c = lax.axis_index("c")
        s = lax.axis_index("s")
        def fill_ones(i, _):
            ones_v[pl.ds(i * LANES, LANES)] = jnp.ones((LANES,), jnp.float32)
            return 0
        lax.fori_loop(0, IDXW // LANES, fill_ones, 0)
        def fill_z(i, _):
            zbuf[pl.ds(i * LANES, LANES)] = jnp.zeros((LANES,), jnp.float32)
            return 0
        lax.fori_loop(0, rps // LANES, fill_z, 0)
        pltpu.sync_copy(zbuf, acc.at[pl.ds(s * rps, rps)])
        plsc.subcore_barrier()
        wid = c * N_SUB + s
        row0 = wid * (epw // IDXW)
        def win(w, _):
            pltpu.sync_copy(dst_hbm.at[pl.ds(row0 + w * 8, 8)], dstv)
            descs = [pltpu.async_copy(ones_v, acc.at[dstv.at[j]], sem, add=True)
                     for j in range(8)]
            for d in descs:
                d.wait()
            return 0
        lax.fori_loop(0, n_win, win, 0)
        plsc.subcore_barrier()
        pltpu.sync_copy(acc.at[pl.ds(s * rps, rps)], zbuf)

        @pl.when(c == 0)
        def _():
            pltpu.sync_copy(zbuf, out0_hbm.at[pl.ds(s * rps, rps)])

        @pl.when(c == 1)
        def _():
            pltpu.sync_copy(zbuf, out1_hbm.at[pl.ds(s * rps, rps)])

    return pl.kernel(
        body,
        out_type=(jax.ShapeDtypeStruct((n_pad,), jnp.float32),
                  jax.ShapeDtypeStruct((n_pad,), jnp.float32)),
        mesh=_sc_mesh(),
        scratch_types=[
            pltpu.VMEM((IDXW,), jnp.float32),
            pltpu.VMEM((8, IDXW), jnp.int32),
            pltpu.VMEM((rps,), jnp.float32),
            pltpu.VMEM_SHARED((n_pad,), jnp.float32),
            pltpu.SemaphoreType.DMA,
        ],
        compiler_params=pltpu.CompilerParams(use_tc_tiling_on_sc=False),
    )


@functools.lru_cache(maxsize=None)
def _make_agg(n_blocks, n_pad, e_pad):
    eps = e_pad // N_SUB
    assert eps % (2 * GIDXW) == 0
    n_pair = eps // (2 * GIDXW)
    rps = n_pad // N_SUB
    assert rps % WCHUNK == 0 and WCHUNK % 8 == 0
    packed_rows = n_pad * n_blocks * LANES // 128

    def body(z_hbm, src_hbm, dst_hbm, out_hbm,
             srcv, dstv, sidx, rows, zbuf, acc, gsem, ssem):
        zf = z_hbm
        of = out_hbm
        c = lax.axis_index("c")
        s = lax.axis_index("s")

        def fill_z(i, _):
            zbuf[i] = jnp.zeros((LANES,), jnp.float32)
            return 0
        lax.fori_loop(0, WCHUNK, fill_z, 0)

        def process(b):
            def zero(k, _):
                pltpu.sync_copy(zbuf, acc.at[pl.ds(s * rps + k * WCHUNK, WCHUNK)])
                return 0
            lax.fori_loop(0, rps // WCHUNK, zero, 0)
            plsc.subcore_barrier()
            row0 = s * (eps // GIDXW)

            def load_and_shift(p, r):
                pltpu.sync_copy(src_hbm.at[pl.ds(r, 1)], srcv.at[pl.ds(p, 1)])
                pltpu.sync_copy(dst_hbm.at[pl.ds(r, 1)], dstv.at[pl.ds(p, 1)])
                for k in range(GIDXW // LANES):
                    sl = pl.ds(k * LANES, LANES)
                    sidx[p, sl] = srcv[p, sl] * n_blocks + b

            def rbuf(p):
                return rows.at[pl.ds(p * GIDXW, GIDXW)]

            def drain_scatter():
                pltpu.make_async_copy(zf.at[pl.ds(0, GIDXW)], rbuf(0),
                                      ssem).wait()

            def pair(t, _):
                r = row0 + 2 * t

                @pl.when(t > 0)
                def _():
                    drain_scatter()
                load_and_shift(0, r)
                g0 = pltpu.async_copy(zf.at[sidx.at[0]], rbuf(0), gsem)

                @pl.when(t > 0)
                def _():
                    drain_scatter()
                load_and_shift(1, r + 1)
                g0.wait()
                pltpu.async_copy(rbuf(0), acc.at[dstv.at[0]], ssem, add=True)
                g1 = pltpu.async_copy(zf.at[sidx.at[1]], rbuf(1), gsem)
                g1.wait()
                pltpu.async_copy(rbuf(1), acc.at[dstv.at[1]], ssem, add=True)
                return 0
            lax.fori_loop(0, n_pair, pair, 0)
            drain_scatter()
            drain_scatter()
            plsc.subcore_barrier()

            def wb(k, _):
                r0 = s * rps + k * WCHUNK
                pltpu.sync_copy(acc.at[pl.ds(r0, WCHUNK)], rows.at[pl.ds(0, WCHUNK)])
                pltpu.sync_copy(rows.at[pl.ds(0, WCHUNK)],
                                of.at[pl.ds(r0, WCHUNK), b])
                return 0
            lax.fori_loop(0, rps // WCHUNK, wb, 0)
            plsc.subcore_barrier()

        for c_val in range(N_CORES):
            blocks = list(range(c_val, n_blocks, N_CORES))
            if not blocks:
                continue

            @pl.when(c == c_val)
            def _(blocks=blocks):
                for b in blocks:
                    process(b)

    del packed_rows
    return pl.kernel(
        body,
        out_type=jax.ShapeDtypeStruct((n_pad, n_blocks, LANES), jnp.float32),
        mesh=_sc_mesh(),
        scratch_types=[
            pltpu.VMEM((2, GIDXW), jnp.int32),
            pltpu.VMEM((2, GIDXW), jnp.int32),
            pltpu.VMEM((2, GIDXW), jnp.int32),
            pltpu.VMEM((2 * GIDXW, LANES), jnp.float32),
            pltpu.VMEM((WCHUNK, LANES), jnp.float32),
            pltpu.VMEM_SHARED((n_pad, LANES), jnp.float32),
            pltpu.SemaphoreType.DMA,
            pltpu.SemaphoreType.DMA,
        ],
        compiler_params=pltpu.CompilerParams(use_tc_tiling_on_sc=False),
    )


def _sc_agg(zp, src2, dst2, nb, e_pad):
    zf = zp.reshape(N_PAD * nb, LANES)
    src2g = src2.reshape(e_pad // GIDXW, GIDXW)
    dst2g = dst2.reshape(e_pad // GIDXW, GIDXW)
    y = _make_agg(nb, N_PAD, e_pad)(zf, src2g, dst2g)
    return y.reshape(N_PAD * nb * LANES // 128, 128)


GRID = N_PAD // ROW_R


def _leaky(v):
    return jnp.where(v >= 0, v, 0.01 * v)


def _pspec(d):
    rows = ROW_R * d // 128
    return pl.BlockSpec((rows, 128), lambda i: (i, 0))


def _rspec(w):
    return pl.BlockSpec((ROW_R, w), lambda i: (i, 0))


def _fspec(shape):
    return pl.BlockSpec(shape, lambda i: (0, 0))


def _packed_struct(d):
    return jax.ShapeDtypeStruct((N_PAD * d // 128, 128), jnp.float32)


def _unpack(ref, d):
    s = 128 // d
    y = ref[...]
    parts = [y[:, q * d:(q + 1) * d].reshape(ROW_R // s, 1, d) for q in range(s)]
    return jnp.concatenate(parts, axis=1).reshape(ROW_R, d)


def _pack(v, d):
    s = 128 // d
    t = v.reshape(ROW_R // s, s, d)
    return jnp.concatenate(
        [t[:, q:q + 1, :].reshape(ROW_R // s, d) for q in range(s)], axis=-1)


def _tc1(xp, W1, d0, d1):
    dW1 = W1.shape[1]

    def body(x_r, d0_r, d1_r, w_r, z_r, dinv_r):
        dinv = lax.rsqrt(d0_r[...] + d1_r[...] + 1.0)
        dinv_r[...] = dinv
        z_r[...] = _pack(dinv * jnp.dot(x_r[...], w_r[...],
                                        preferred_element_type=jnp.float32), dW1)

    return pl.pallas_call(
        body, grid=(GRID,),
        in_specs=[_rspec(xp.shape[1]), _rspec(1), _rspec(1), _fspec(W1.shape)],
        out_specs=[_pspec(dW1), _rspec(1)],
        out_shape=[_packed_struct(dW1),
                   jax.ShapeDtypeStruct((N_PAD, 1), jnp.float32)],
    )(xp, d0, d1, W1)


def _tc2(y1, z1, dinv, b1, d):
    def body(y_r, z_r, dv_r, b_r, o_r):
        h = _leaky(dv_r[...] * (_unpack(y_r, d) + _unpack(z_r, d)) + b_r[...])
        o_r[...] = _pack(dv_r[...] * h, d)

    return pl.pallas_call(
        body, grid=(GRID,),
        in_specs=[_pspec(d), _pspec(d), _rspec(1), _fspec(b1.shape)],
        out_specs=_pspec(d), out_shape=_packed_struct(d),
    )(y1, z1, dinv, b1)


def _tc3(y2, z2, dinv, W2, b2, W3, d_in):
    d_out = W3.shape[1]

    def body(y_r, z_r, dv_r, w2_r, b2_r, w3_r, o_r):
        t = dv_r[...] * (_unpack(y_r, d_in) + _unpack(z_r, d_in))
        h = _leaky(jnp.dot(t, w2_r[...], preferred_element_type=jnp.float32)
                   + b2_r[...])
        o_r[...] = _pack(dv_r[...] * jnp.dot(h, w3_r[...],
                                             preferred_element_type=jnp.float32),
                         d_out)

    return pl.pallas_call(
        body, grid=(GRID,),
        in_specs=[_pspec(d_in), _pspec(d_in), _rspec(1),
                  _fspec(W2.shape), _fspec(b2.shape), _fspec(W3.shape)],
        out_specs=_pspec(d_out), out_shape=_packed_struct(d_out),
    )(y2, z2, dinv, W2, b2, W3)


def _tc4(y3, z3, dinv, b3, W4, d_in):
    d_out = W4.shape[1]

    def body(y_r, z_r, dv_r, b3_r, w4_r, o_r):
        h = _leaky(dv_r[...] * (_unpack(y_r, d_in) + _unpack(z_r, d_in))
                   + b3_r[...])
        o_r[...] = _pack(dv_r[...] * jnp.dot(h, w4_r[...],
                                             preferred_element_type=jnp.float32),
                         d_out)

    return pl.pallas_call(
        body, grid=(GRID,),
        in_specs=[_pspec(d_in), _pspec(d_in), _rspec(1),
                  _fspec(b3.shape), _fspec(W4.shape)],
        out_specs=_pspec(d_out), out_shape=_packed_struct(d_out),
    )(y3, z3, dinv, b3, W4)


def _tc5(y4, z4, dinv, b4, Wl, bl, d_in):
    d_out = Wl.shape[1]

    def body(y_r, z_r, dv_r, b4_r, wl_r, bl_r, o_r):
        t = (dv_r[...] * (_unpack(y_r, d_in) + _unpack(z_r, d_in)))
        h = _leaky(t[:, :b4_r.shape[1]] + b4_r[...])
        o_r[...] = jnp.maximum(
            jnp.dot(h, wl_r[...], preferred_element_type=jnp.float32)
            + bl_r[...], 0.0)

    return pl.pallas_call(
        body, grid=(GRID,),
        in_specs=[_pspec(d_in), _pspec(d_in), _rspec(1),
                  _fspec(b4.shape), _fspec(Wl.shape), _fspec(bl.shape)],
        out_specs=_rspec(d_out),
        out_shape=jax.ShapeDtypeStruct((N_PAD, d_out), jnp.float32),
    )(y4, z4, dinv, b4, Wl, bl)


def kernel(x, edge_index, W1, b1, W2, b2, W3, b3, W4, b4, Wl, bl):
    n = x.shape[0]
    e = edge_index.shape[1]
    assert n <= N_PAD

    e_unit = N_CORES * N_SUB * STREAMS * IDXW
    e_pad = ((e + e_unit - 1) // e_unit) * e_unit

    src = edge_index[0].astype(jnp.int32)
    dst = edge_index[1].astype(jnp.int32)
    if e_pad != e:
        pad = e_pad - e
        pad_dst = n + jnp.arange(pad, dtype=jnp.int32) % (N_PAD - n)
        src = jnp.concatenate([src, jnp.full((pad,), n - 1, jnp.int32)])
        dst = jnp.concatenate([dst, pad_dst])
    src2 = src.reshape(e_pad // IDXW, IDXW)
    dst2 = dst.reshape(e_pad // IDXW, IDXW)

    deg0, deg1 = _make_deg(N_PAD, e_pad)(dst2)
    xp = jnp.pad(x, ((0, N_PAD - n), (0, 0)))

    b1r, b2r, b3r, b4r, blr = (v.reshape(1, -1) for v in (b1, b2, b3, b4, bl))

    W4p = jnp.pad(W4, ((0, 0), (0, 64 - W4.shape[1])))

    z1, dinv = _tc1(xp, W1, deg0[:, None], deg1[:, None])
    y1 = _sc_agg(z1, src2, dst2, 2, e_pad)
    z2 = _tc2(y1, z1, dinv, b1r, 32)
    y2 = _sc_agg(z2, src2, dst2, 2, e_pad)
    z3 = _tc3(y2, z2, dinv, W2, b2r, W3, 32)
    y3 = _sc_agg(z3, src2, dst2, 4, e_pad)
    z4 = _tc4(y3, z3, dinv, b3r, W4p, 64)
    y4 = _sc_agg(z4, src2, dst2, 4, e_pad)
    return _tc5(y4, z4, dinv, b4r, Wl, blr, 64)[:n]


# speedup vs baseline: 1.1878x; 1.1622x over previous
import functools

import jax
import jax.numpy as jnp
from jax import lax
from jax.experimental import pallas as pl
from jax.experimental.pallas import tpu as pltpu
from jax.experimental.pallas import tpu_sc as plsc

N_CORES = 2
N_SUB = 16
LANES = 16
IDXW = 128
GIDXW = 512
STREAMS = 8
WCHUNK = 368
N_PAD = 100096
ROW_R = 4352


def _sc_mesh():
    return plsc.VectorSubcoreMesh(
        core_axis_name="c", subcore_axis_name="s",
        num_cores=N_CORES, num_subcores=N_SUB)


@functools.lru_cache(maxsize=None)
def _make_deg(n_pad, e_pad):
    epw = e_pad // (N_CORES * N_SUB)
    assert epw % (8 * IDXW) == 0
    n_win = epw // (8 * IDXW)
    rps = n_pad // N_SUB
    assert rps % 8 == 0 and n_pad % N_SUB == 0

    def body(dst_hbm, out0_hbm, out1_hbm, ones_v, dstv, zbuf, acc, sem):
        c = lax.axis_index("c")
        s = lax.axis_index("s")
        def fill_ones(i, _):
            ones_v[pl.ds(i * LANES, LANES)] = jnp.ones((LANES,), jnp.float32)
            return 0
        lax.fori_loop(0, IDXW // LANES, fill_ones, 0)
        def fill_z(i, _):
            zbuf[pl.ds(i * LANES, LANES)] = jnp.zeros((LANES,), jnp.float32)
            return 0
        lax.fori_loop(0, rps // LANES, fill_z, 0)
        pltpu.sync_copy(zbuf, acc.at[pl.ds(s * rps, rps)])
        plsc.subcore_barrier()
        wid = c * N_SUB + s
        row0 = wid * (epw // IDXW)
        def win(w, _):
            pltpu.sync_copy(dst_hbm.at[pl.ds(row0 + w * 8, 8)], dstv)
            descs = [pltpu.async_copy(ones_v, acc.at[dstv.at[j]], sem, add=True)
                     for j in range(8)]
            for d in descs:
                d.wait()
            return 0
        lax.fori_loop(0, n_win, win, 0)
        plsc.subcore_barrier()
        pltpu.sync_copy(acc.at[pl.ds(s * rps, rps)], zbuf)

        @pl.when(c == 0)
        def _():
            pltpu.sync_copy(zbuf, out0_hbm.at[pl.ds(s * rps, rps)])

        @pl.when(c == 1)
        def _():
            pltpu.sync_copy(zbuf, out1_hbm.at[pl.ds(s * rps, rps)])

    return pl.kernel(
        body,
        out_type=(jax.ShapeDtypeStruct((n_pad,), jnp.float32),
                  jax.ShapeDtypeStruct((n_pad,), jnp.float32)),
        mesh=_sc_mesh(),
        scratch_types=[
            pltpu.VMEM((IDXW,), jnp.float32),
            pltpu.VMEM((8, IDXW), jnp.int32),
            pltpu.VMEM((rps,), jnp.float32),
            pltpu.VMEM_SHARED((n_pad,), jnp.float32),
            pltpu.SemaphoreType.DMA,
        ],
        compiler_params=pltpu.CompilerParams(use_tc_tiling_on_sc=False),
    )


@functools.lru_cache(maxsize=None)
def _make_agg(n_blocks, n_pad, e_pad):
    eps = e_pad // N_SUB
    assert eps % (2 * GIDXW) == 0
    n_pair = eps // (2 * GIDXW)
    rps = n_pad // N_SUB
    assert rps % WCHUNK == 0 and WCHUNK % 8 == 0
    packed_rows = n_pad * n_blocks * LANES // 128

    def body(z_hbm, src_hbm, dst_hbm, out_hbm,
             srcv, dstv, sidx, rows, zbuf, acc, gsem, ssem):
        zf = z_hbm
        of = out_hbm
        c = lax.axis_index("c")
        s = lax.axis_index("s")

        def fill_z(i, _):
            zbuf[i] = jnp.zeros((LANES,), jnp.float32)
            return 0
        lax.fori_loop(0, WCHUNK, fill_z, 0)

        def process(b):
            def zero(k, _):
                pltpu.sync_copy(zbuf, acc.at[pl.ds(s * rps + k * WCHUNK, WCHUNK)])
                return 0
            lax.fori_loop(0, rps // WCHUNK, zero, 0)
            plsc.subcore_barrier()
            row0 = s * (eps // GIDXW)

            def load_and_shift(p, r):
                pltpu.sync_copy(src_hbm.at[pl.ds(r, 1)], srcv.at[pl.ds(p, 1)])
                pltpu.sync_copy(dst_hbm.at[pl.ds(r, 1)], dstv.at[pl.ds(p, 1)])
                for k in range(GIDXW // LANES):
                    sl = pl.ds(k * LANES, LANES)
                    sidx[p, sl] = srcv[p, sl] * n_blocks + b

            def rbuf(p):
                return rows.at[pl.ds(p * GIDXW, GIDXW)]

            def drain_scatter():
                pltpu.make_async_copy(zf.at[pl.ds(0, GIDXW)], rbuf(0),
                                      ssem).wait()

            def pair(t, _):
                r = row0 + 2 * t

                @pl.when(t > 0)
                def _():
                    drain_scatter()
                load_and_shift(0, r)
                g0 = pltpu.async_copy(zf.at[sidx.at[0]], rbuf(0), gsem)

                @pl.when(t > 0)
                def _():
                    drain_scatter()
                load_and_shift(1, r + 1)
                g0.wait()
                pltpu.async_copy(rbuf(0), acc.at[dstv.at[0]], ssem, add=True)
                g1 = pltpu.async_copy(zf.at[sidx.at[1]], rbuf(1), gsem)
                g1.wait()
                pltpu.async_copy(rbuf(1), acc.at[dstv.at[1]], ssem, add=True)
                return 0
            lax.fori_loop(0, n_pair, pair, 0)
            drain_scatter()
            drain_scatter()
            plsc.subcore_barrier()

            def wb(k, _):
                r0 = s * rps + k * WCHUNK
                pltpu.sync_copy(acc.at[pl.ds(r0, WCHUNK)], rows.at[pl.ds(0, WCHUNK)])
                pltpu.sync_copy(rows.at[pl.ds(0, WCHUNK)],
                                of.at[pl.ds(r0, WCHUNK), b])
                return 0
            lax.fori_loop(0, rps // WCHUNK, wb, 0)
            plsc.subcore_barrier()

        for c_val in range(N_CORES):
            blocks = list(range(c_val, n_blocks, N_CORES))
            if not blocks:
                continue

            @pl.when(c == c_val)
            def _(blocks=blocks):
                for b in blocks:
                    process(b)

    del packed_rows
    return pl.kernel(
        body,
        out_type=jax.ShapeDtypeStruct((n_pad, n_blocks, LANES), jnp.float32),
        mesh=_sc_mesh(),
        scratch_types=[
            pltpu.VMEM((2, GIDXW), jnp.int32),
            pltpu.VMEM((2, GIDXW), jnp.int32),
            pltpu.VMEM((2, GIDXW), jnp.int32),
            pltpu.VMEM((2 * GIDXW, LANES), jnp.float32),
            pltpu.VMEM((WCHUNK, LANES), jnp.float32),
            pltpu.VMEM_SHARED((n_pad, LANES), jnp.float32),
            pltpu.SemaphoreType.DMA,
            pltpu.SemaphoreType.DMA,
        ],
        compiler_params=pltpu.CompilerParams(use_tc_tiling_on_sc=False),
    )


def _sc_agg(zp, src2, dst2, nb, e_pad):
    zf = zp.reshape(N_PAD * nb, LANES)
    src2g = src2.reshape(e_pad // GIDXW, GIDXW)
    dst2g = dst2.reshape(e_pad // GIDXW, GIDXW)
    y = _make_agg(nb, N_PAD, e_pad)(zf, src2g, dst2g)
    return y.reshape(N_PAD * nb * LANES // 128, 128)


GRID = N_PAD // ROW_R


def _leaky(v):
    return jnp.where(v >= 0, v, 0.01 * v)


def _pspec(d):
    rows = ROW_R * d // 128
    return pl.BlockSpec((rows, 128), lambda i: (i, 0))


def _rspec(w):
    return pl.BlockSpec((ROW_R, w), lambda i: (i, 0))


def _fspec(shape):
    return pl.BlockSpec(shape, lambda i: (0, 0))


def _packed_struct(d):
    return jax.ShapeDtypeStruct((N_PAD * d // 128, 128), jnp.float32)


def _unpack(ref, d):
    s = 128 // d
    y = ref[...]
    parts = [y[:, q * d:(q + 1) * d].reshape(ROW_R // s, 1, d) for q in range(s)]
    return jnp.concatenate(parts, axis=1).reshape(ROW_R, d)


def _pack(v, d):
    s = 128 // d
    t = v.reshape(ROW_R // s, s, d)
    return jnp.concatenate(
        [t[:, q:q + 1, :].reshape(ROW_R // s, d) for q in range(s)], axis=-1)


def _tc1(xp, W1, d0, d1):
    dW1 = W1.shape[1]

    def body(x_r, d0_r, d1_r, w_r, z_r, dp32_r, dp64_r):
        dinv = lax.rsqrt(d0_r[...] + d1_r[...] + 1.0)
        dp32_r[...] = _pack(jnp.broadcast_to(dinv, (ROW_R, 32)), 32)
        dp64_r[...] = _pack(jnp.broadcast_to(dinv, (ROW_R, 64)), 64)
        z_r[...] = _pack(dinv * jnp.dot(x_r[...], w_r[...],
                                        preferred_element_type=jnp.float32), dW1)

    return pl.pallas_call(
        body, grid=(GRID,),
        in_specs=[_rspec(xp.shape[1]), _rspec(1), _rspec(1), _fspec(W1.shape)],
        out_specs=[_pspec(dW1), _pspec(32), _pspec(64)],
        out_shape=[_packed_struct(dW1), _packed_struct(32), _packed_struct(64)],
    )(xp, d0, d1, W1)


def _tc2(y1, z1, dp32, b1p):
    def body(y_r, z_r, dp_r, b_r, o_r):
        dp = dp_r[...]
        o_r[...] = dp * _leaky(dp * (y_r[...] + z_r[...]) + b_r[...])

    return pl.pallas_call(
        body, grid=(GRID,),
        in_specs=[_pspec(32), _pspec(32), _pspec(32), _fspec(b1p.shape)],
        out_specs=_pspec(32), out_shape=_packed_struct(32),
    )(y1, z1, dp32, b1p)


def _tc3(y2, z2, dp32, dp64, W2, b2, W3):
    def body(y_r, z_r, dp_r, dq_r, w2_r, b2_r, w3_r, o_r):
        t = _unpack(dp_r[...] * (y_r[...] + z_r[...]), 32)
        h = _leaky(jnp.dot(t, w2_r[...], preferred_element_type=jnp.float32)
                   + b2_r[...])
        o_r[...] = dq_r[...] * _pack(
            jnp.dot(h, w3_r[...], preferred_element_type=jnp.float32), 64)

    return pl.pallas_call(
        body, grid=(GRID,),
        in_specs=[_pspec(32), _pspec(32), _pspec(32), _pspec(64),
                  _fspec(W2.shape), _fspec(b2.shape), _fspec(W3.shape)],
        out_specs=_pspec(64), out_shape=_packed_struct(64),
    )(y2, z2, dp32, dp64, W2, b2, W3)


def _tc4(y3, z3, dp64, b3p, W4p):
    def body(y_r, z_r, dp_r, b3_r, w4_r, o_r):
        dp = dp_r[...]
        h = _unpack(_leaky(dp * (y_r[...] + z_r[...]) + b3_r[...]), 64)
        o_r[...] = dp * _pack(
            jnp.dot(h, w4_r[...], preferred_element_type=jnp.float32), 64)

    return pl.pallas_call(
        body, grid=(GRID,),
        in_specs=[_pspec(64), _pspec(64), _pspec(64),
                  _fspec(b3p.shape), _fspec(W4p.shape)],
        out_specs=_pspec(64), out_shape=_packed_struct(64),
    )(y3, z3, dp64, b3p, W4p)


def _tc5(y4, z4, dp64, b4p, Wl, bl):
    d_out = Wl.shape[1]

    def body(y_r, z_r, dp_r, b4_r, wl_r, bl_r, o_r):
        h_p = _leaky(dp_r[...] * (y_r[...] + z_r[...]) + b4_r[...])
        h = _unpack(h_p, 64)[:, :Wl.shape[0]]
        o_r[...] = jnp.maximum(
            jnp.dot(h, wl_r[...], preferred_element_type=jnp.float32)
            + bl_r[...], 0.0)

    return pl.pallas_call(
        body, grid=(GRID,),
        in_specs=[_pspec(64), _pspec(64), _pspec(64),
                  _fspec(b4p.shape), _fspec(Wl.shape), _fspec(bl.shape)],
        out_specs=_rspec(d_out),
        out_shape=jax.ShapeDtypeStruct((N_PAD, d_out), jnp.float32),
    )(y4, z4, dp64, b4p, Wl, bl)


def kernel(x, edge_index, W1, b1, W2, b2, W3, b3, W4, b4, Wl, bl):
    n = x.shape[0]
    e = edge_index.shape[1]
    assert n <= N_PAD

    e_unit = N_CORES * N_SUB * STREAMS * IDXW
    e_pad = ((e + e_unit - 1) // e_unit) * e_unit

    src = edge_index[0].astype(jnp.int32)
    dst = edge_index[1].astype(jnp.int32)
    if e_pad != e:
        pad = e_pad - e
        pad_dst = n + jnp.arange(pad, dtype=jnp.int32) % (N_PAD - n)
        src = jnp.concatenate([src, jnp.full((pad,), n - 1, jnp.int32)])
        dst = jnp.concatenate([dst, pad_dst])
    src2 = src.reshape(e_pad // IDXW, IDXW)
    dst2 = dst.reshape(e_pad // IDXW, IDXW)

    deg0, deg1 = _make_deg(N_PAD, e_pad)(dst2)
    xp = jnp.pad(x, ((0, N_PAD - n), (0, 0)))

    b2r, blr = b2.reshape(1, -1), bl.reshape(1, -1)
    b1p = jnp.tile(b1, 4).reshape(1, 128)
    b3p = jnp.tile(b3, 2).reshape(1, 128)
    b4p = jnp.tile(jnp.concatenate([b4, jnp.zeros((16,), jnp.float32)]),
                   2).reshape(1, 128)

    W4p = jnp.pad(W4, ((0, 0), (0, 64 - W4.shape[1])))

    z1, dp32, dp64 = _tc1(xp, W1, deg0[:, None], deg1[:, None])
    y1 = _sc_agg(z1, src2, dst2, 2, e_pad)
    z2 = _tc2(y1, z1, dp32, b1p)
    y2 = _sc_agg(z2, src2, dst2, 2, e_pad)
    z3 = _tc3(y2, z2, dp32, dp64, W2, b2r, W3)
    y3 = _sc_agg(z3, src2, dst2, 4, e_pad)
    z4 = _tc4(y3, z3, dp64, b3p, W4p)
    y4 = _sc_agg(z4, src2, dst2, 4, e_pad)
    return _tc5(y4, z4, dp64, b4p, Wl, blr)[:n]


# speedup vs baseline: 1.4112x; 1.1881x over previous
import functools

import jax
import jax.numpy as jnp
from jax import lax
from jax.experimental import pallas as pl
from jax.experimental.pallas import tpu as pltpu
from jax.experimental.pallas import tpu_sc as plsc

N_CORES = 2
N_SUB = 16
LANES = 16
IDXW = 128
GIDXW = 512
STREAMS = 8
WCHUNK = 368
N_PAD = 100096
ROW_R = 4352


def _sc_mesh():
    return plsc.VectorSubcoreMesh(
        core_axis_name="c", subcore_axis_name="s",
        num_cores=N_CORES, num_subcores=N_SUB)


@functools.lru_cache(maxsize=None)
def _make_deg(n_pad, e_pad):
    epw = e_pad // (N_CORES * N_SUB)
    assert epw % (8 * IDXW) == 0
    n_win = epw // (8 * IDXW)
    rps = n_pad // N_SUB
    assert rps % 8 == 0 and n_pad % N_SUB == 0

    def body(dst_hbm, out0_hbm, out1_hbm, ones_v, dstv, zbuf, acc, sem):
        c = lax.axis_index("c")
        s = lax.axis_index("s")
        def fill_ones(i, _):
            ones_v[pl.ds(i * LANES, LANES)] = jnp.ones((LANES,), jnp.float32)
            return 0
        lax.fori_loop(0, IDXW // LANES, fill_ones, 0)
        def fill_z(i, _):
            zbuf[pl.ds(i * LANES, LANES)] = jnp.zeros((LANES,), jnp.float32)
            return 0
        lax.fori_loop(0, rps // LANES, fill_z, 0)
        pltpu.sync_copy(zbuf, acc.at[pl.ds(s * rps, rps)])
        plsc.subcore_barrier()
        wid = c * N_SUB + s
        row0 = wid * (epw // IDXW)
        def win(w, _):
            pltpu.sync_copy(dst_hbm.at[pl.ds(row0 + w * 8, 8)], dstv)
            descs = [pltpu.async_copy(ones_v, acc.at[dstv.at[j]], sem, add=True)
                     for j in range(8)]
            for d in descs:
                d.wait()
            return 0
        lax.fori_loop(0, n_win, win, 0)
        plsc.subcore_barrier()
        pltpu.sync_copy(acc.at[pl.ds(s * rps, rps)], zbuf)

        @pl.when(c == 0)
        def _():
            pltpu.sync_copy(zbuf, out0_hbm.at[pl.ds(s * rps, rps)])

        @pl.when(c == 1)
        def _():
            pltpu.sync_copy(zbuf, out1_hbm.at[pl.ds(s * rps, rps)])

    return pl.kernel(
        body,
        out_type=(jax.ShapeDtypeStruct((n_pad,), jnp.float32),
                  jax.ShapeDtypeStruct((n_pad,), jnp.float32)),
        mesh=_sc_mesh(),
        scratch_types=[
            pltpu.VMEM((IDXW,), jnp.float32),
            pltpu.VMEM((8, IDXW), jnp.int32),
            pltpu.VMEM((rps,), jnp.float32),
            pltpu.VMEM_SHARED((n_pad,), jnp.float32),
            pltpu.SemaphoreType.DMA,
        ],
        compiler_params=pltpu.CompilerParams(use_tc_tiling_on_sc=False),
    )


@functools.lru_cache(maxsize=None)
def _make_agg(n_blocks, n_pad, e_pad):
    eps = e_pad // N_SUB
    assert eps % (4 * GIDXW) == 0
    n_pair = eps // (2 * GIDXW)
    rps = n_pad // N_SUB
    assert rps % WCHUNK == 0 and WCHUNK % 8 == 0
    packed_rows = n_pad * n_blocks * LANES // 128

    def body(z_hbm, src_hbm, dst_hbm, out_hbm,
             srcv, dstv, sidx, rows, zbuf, acc, gsem, ssem, isem):
        zf = z_hbm
        of = out_hbm
        c = lax.axis_index("c")
        s = lax.axis_index("s")

        def fill_z(i, _):
            zbuf[i] = jnp.zeros((LANES,), jnp.float32)
            return 0
        lax.fori_loop(0, WCHUNK, fill_z, 0)

        def process(b):
            def zero(k, _):
                pltpu.sync_copy(zbuf, acc.at[pl.ds(s * rps + k * WCHUNK, WCHUNK)])
                return 0
            lax.fori_loop(0, rps // WCHUNK, zero, 0)
            plsc.subcore_barrier()
            row0 = s * (eps // GIDXW)
            n_win = 2 * n_pair

            def idx_load(w, slot):
                pltpu.async_copy(src_hbm.at[pl.ds(row0 + w, 1)],
                                 srcv.at[pl.ds(slot, 1)], isem)
                pltpu.async_copy(dst_hbm.at[pl.ds(row0 + w, 1)],
                                 dstv.at[pl.ds(slot, 1)], isem)

            def rbuf(p):
                return rows.at[pl.ds(p * GIDXW, GIDXW)]

            def drain_scatter():
                pltpu.make_async_copy(zf.at[pl.ds(0, GIDXW)], rbuf(0),
                                      ssem).wait()

            def drain_idx(slot):
                for ref in (srcv, dstv):
                    pltpu.make_async_copy(src_hbm.at[pl.ds(0, 1)],
                                          ref.at[pl.ds(slot, 1)], isem).wait()

            idx_load(0, 0)
            idx_load(1, 1)

            def quad(t, _):
                for q in range(4):
                    w = 4 * t + q
                    slot, rslot = q, q % 2
                    if q >= 2:
                        drain_scatter()
                    else:
                        @pl.when(t > 0)
                        def _():
                            drain_scatter()

                    @pl.when(w + 2 < n_win)
                    def _():
                        idx_load(w + 2, (q + 2) % 4)
                    drain_idx(slot)
                    for k in range(GIDXW // LANES):
                        sl = pl.ds(k * LANES, LANES)
                        sidx[slot, sl] = srcv[slot, sl] * n_blocks + b
                    pltpu.async_copy(zf.at[sidx.at[slot]], rbuf(rslot),
                                     gsem).wait()
                    pltpu.async_copy(rbuf(rslot), acc.at[dstv.at[slot]],
                                     ssem, add=True)
                return 0
            lax.fori_loop(0, n_win // 4, quad, 0)
            drain_scatter()
            drain_scatter()
            plsc.subcore_barrier()

            def wb(k, _):
                r0 = s * rps + k * WCHUNK
                pltpu.sync_copy(acc.at[pl.ds(r0, WCHUNK)], rows.at[pl.ds(0, WCHUNK)])
                pltpu.sync_copy(rows.at[pl.ds(0, WCHUNK)],
                                of.at[pl.ds(r0, WCHUNK), b])
                return 0
            lax.fori_loop(0, rps // WCHUNK, wb, 0)
            plsc.subcore_barrier()

        for c_val in range(N_CORES):
            blocks = list(range(c_val, n_blocks, N_CORES))
            if not blocks:
                continue

            @pl.when(c == c_val)
            def _(blocks=blocks):
                for b in blocks:
                    process(b)

    del packed_rows
    return pl.kernel(
        body,
        out_type=jax.ShapeDtypeStruct((n_pad, n_blocks, LANES), jnp.float32),
        mesh=_sc_mesh(),
        scratch_types=[
            pltpu.VMEM((4, GIDXW), jnp.int32),
            pltpu.VMEM((4, GIDXW), jnp.int32),
            pltpu.VMEM((4, GIDXW), jnp.int32),
            pltpu.VMEM((2 * GIDXW, LANES), jnp.float32),
            pltpu.VMEM((WCHUNK, LANES), jnp.float32),
            pltpu.VMEM_SHARED((n_pad, LANES), jnp.float32),
            pltpu.SemaphoreType.DMA,
            pltpu.SemaphoreType.DMA,
            pltpu.SemaphoreType.DMA,
        ],
        compiler_params=pltpu.CompilerParams(use_tc_tiling_on_sc=False),
    )


def _sc_agg(zp, src2, dst2, nb, e_pad):
    zf = zp.reshape(N_PAD * nb, LANES)
    src2g = src2.reshape(e_pad // GIDXW, GIDXW)
    dst2g = dst2.reshape(e_pad // GIDXW, GIDXW)
    y = _make_agg(nb, N_PAD, e_pad)(zf, src2g, dst2g)
    return y.reshape(N_PAD * nb * LANES // 128, 128)


GRID = N_PAD // ROW_R


def _leaky(v):
    return jnp.where(v >= 0, v, 0.01 * v)


def _pspec(d):
    rows = ROW_R * d // 128
    return pl.BlockSpec((rows, 128), lambda i: (i, 0))


def _rspec(w):
    return pl.BlockSpec((ROW_R, w), lambda i: (i, 0))


def _fspec(shape):
    return pl.BlockSpec(shape, lambda i: (0, 0))


def _packed_struct(d):
    return jax.ShapeDtypeStruct((N_PAD * d // 128, 128), jnp.float32)


def _unpack(ref, d):
    s = 128 // d
    y = ref[...]
    parts = [y[:, q * d:(q + 1) * d].reshape(ROW_R // s, 1, d) for q in range(s)]
    return jnp.concatenate(parts, axis=1).reshape(ROW_R, d)


def _pack(v, d):
    s = 128 // d
    t = v.reshape(ROW_R // s, s, d)
    return jnp.concatenate(
        [t[:, q:q + 1, :].reshape(ROW_R // s, d) for q in range(s)], axis=-1)


def _tc1(xp, W1, d0, d1):
    dW1 = W1.shape[1]

    def body(x_r, d0_r, d1_r, w_r, z_r, dp32_r, dp64_r):
        dinv = lax.rsqrt(d0_r[...] + d1_r[...] + 1.0)
        dp32_r[...] = _pack(jnp.broadcast_to(dinv, (ROW_R, 32)), 32)
        dp64_r[...] = _pack(jnp.broadcast_to(dinv, (ROW_R, 64)), 64)
        z_r[...] = _pack(dinv * jnp.dot(x_r[...], w_r[...],
                                        preferred_element_type=jnp.float32), dW1)

    return pl.pallas_call(
        body, grid=(GRID,),
        in_specs=[_rspec(xp.shape[1]), _rspec(1), _rspec(1), _fspec(W1.shape)],
        out_specs=[_pspec(dW1), _pspec(32), _pspec(64)],
        out_shape=[_packed_struct(dW1), _packed_struct(32), _packed_struct(64)],
    )(xp, d0, d1, W1)


def _tc2(y1, z1, dp32, b1p):
    def body(y_r, z_r, dp_r, b_r, o_r):
        dp = dp_r[...]
        o_r[...] = dp * _leaky(dp * (y_r[...] + z_r[...]) + b_r[...])

    return pl.pallas_call(
        body, grid=(GRID,),
        in_specs=[_pspec(32), _pspec(32), _pspec(32), _fspec(b1p.shape)],
        out_specs=_pspec(32), out_shape=_packed_struct(32),
    )(y1, z1, dp32, b1p)


def _tc3(y2, z2, dp32, dp64, W2, b2, W3):
    def body(y_r, z_r, dp_r, dq_r, w2_r, b2_r, w3_r, o_r):
        t = _unpack(dp_r[...] * (y_r[...] + z_r[...]), 32)
        h = _leaky(jnp.dot(t, w2_r[...], preferred_element_type=jnp.float32)
                   + b2_r[...])
        o_r[...] = dq_r[...] * _pack(
            jnp.dot(h, w3_r[...], preferred_element_type=jnp.float32), 64)

    return pl.pallas_call(
        body, grid=(GRID,),
        in_specs=[_pspec(32), _pspec(32), _pspec(32), _pspec(64),
                  _fspec(W2.shape), _fspec(b2.shape), _fspec(W3.shape)],
        out_specs=_pspec(64), out_shape=_packed_struct(64),
    )(y2, z2, dp32, dp64, W2, b2, W3)


def _tc4(y3, z3, dp64, b3p, W4p):
    def body(y_r, z_r, dp_r, b3_r, w4_r, o_r):
        dp = dp_r[...]
        h = _unpack(_leaky(dp * (y_r[...] + z_r[...]) + b3_r[...]), 64)
        o_r[...] = dp * _pack(
            jnp.dot(h, w4_r[...], preferred_element_type=jnp.float32), 64)

    return pl.pallas_call(
        body, grid=(GRID,),
        in_specs=[_pspec(64), _pspec(64), _pspec(64),
                  _fspec(b3p.shape), _fspec(W4p.shape)],
        out_specs=_pspec(64), out_shape=_packed_struct(64),
    )(y3, z3, dp64, b3p, W4p)


def _tc5(y4, z4, dp64, b4p, Wl, bl):
    d_out = Wl.shape[1]

    def body(y_r, z_r, dp_r, b4_r, wl_r, bl_r, o_r):
        h_p = _leaky(dp_r[...] * (y_r[...] + z_r[...]) + b4_r[...])
        h = _unpack(h_p, 64)[:, :Wl.shape[0]]
        o_r[...] = jnp.maximum(
            jnp.dot(h, wl_r[...], preferred_element_type=jnp.float32)
            + bl_r[...], 0.0)

    return pl.pallas_call(
        body, grid=(GRID,),
        in_specs=[_pspec(64), _pspec(64), _pspec(64),
                  _fspec(b4p.shape), _fspec(Wl.shape), _fspec(bl.shape)],
        out_specs=_rspec(d_out),
        out_shape=jax.ShapeDtypeStruct((N_PAD, d_out), jnp.float32),
    )(y4, z4, dp64, b4p, Wl, bl)


def kernel(x, edge_index, W1, b1, W2, b2, W3, b3, W4, b4, Wl, bl):
    n = x.shape[0]
    e = edge_index.shape[1]
    assert n <= N_PAD

    e_unit = N_CORES * N_SUB * STREAMS * IDXW
    e_pad = ((e + e_unit - 1) // e_unit) * e_unit

    src = edge_index[0].astype(jnp.int32)
    dst = edge_index[1].astype(jnp.int32)
    if e_pad != e:
        pad = e_pad - e
        pad_dst = n + jnp.arange(pad, dtype=jnp.int32) % (N_PAD - n)
        src = jnp.concatenate([src, jnp.full((pad,), n - 1, jnp.int32)])
        dst = jnp.concatenate([dst, pad_dst])
    src2 = src.reshape(e_pad // IDXW, IDXW)
    dst2 = dst.reshape(e_pad // IDXW, IDXW)

    deg0, deg1 = _make_deg(N_PAD, e_pad)(dst2)
    xp = jnp.pad(x, ((0, N_PAD - n), (0, 0)))

    b2r, blr = b2.reshape(1, -1), bl.reshape(1, -1)
    b1p = jnp.tile(b1, 4).reshape(1, 128)
    b3p = jnp.tile(b3, 2).reshape(1, 128)
    b4p = jnp.tile(jnp.concatenate([b4, jnp.zeros((16,), jnp.float32)]),
                   2).reshape(1, 128)

    W4p = jnp.pad(W4, ((0, 0), (0, 64 - W4.shape[1])))

    z1, dp32, dp64 = _tc1(xp, W1, deg0[:, None], deg1[:, None])
    y1 = _sc_agg(z1, src2, dst2, 2, e_pad)
    z2 = _tc2(y1, z1, dp32, b1p)
    y2 = _sc_agg(z2, src2, dst2, 2, e_pad)
    z3 = _tc3(y2, z2, dp32, dp64, W2, b2r, W3)
    y3 = _sc_agg(z3, src2, dst2, 4, e_pad)
    z4 = _tc4(y3, z3, dp64, b3p, W4p)
    y4 = _sc_agg(z4, src2, dst2, 4, e_pad)
    return _tc5(y4, z4, dp64, b4p, Wl, blr)[:n]
